# baseline (speedup 1.0000x reference)
import jax
import jax.numpy as jnp
from jax import lax
from jax.experimental import pallas as pl
from jax.experimental.pallas import tpu as pltpu
from jax.experimental.pallas import tpu_sc as plsc

N_USERS = 5000
N_ITEMS = 5000
N = N_USERS + N_ITEMS
H = 128
E = 320000

NC = 2
NS = 16
NW = NC * NS
CHUNK = 128
CH = 80
EP = NW * CH * CHUNK
NPAD = 10240
ZROWS = 128
RPS = NPAD // NS


K = 16
NB = CH // K


def _spmm_body(x_hbm, src_hbm, dst_hbm, w_hbm, out_hbm,
               srcb0, srcb1, dstb0, dstb1, wb0, wb1,
               rows0, rows1, accum, sem0, sem1, semi):
    c = lax.axis_index("c")
    s = lax.axis_index("s")
    wid = s * NC + c

    idxbufs = ((srcb0, dstb0, wb0), (srcb1, dstb1, wb1))
    rowbufs = ((rows0, sem0), (rows1, sem1))

    def fire_idx(t, bufs):
        pltpu.async_copy(src_hbm.at[wid, pl.ds(t * K, K)], bufs[0], semi)
        pltpu.async_copy(dst_hbm.at[wid, pl.ds(t * K, K)], bufs[1], semi)
        pltpu.async_copy(w_hbm.at[wid, pl.ds(t * K, K)], bufs[2], semi)

    def drain_idx(t, bufs):
        pltpu.make_async_copy(src_hbm.at[wid, pl.ds(t * K, K)], bufs[0], semi).wait()
        pltpu.make_async_copy(dst_hbm.at[wid, pl.ds(t * K, K)], bufs[1], semi).wait()
        pltpu.make_async_copy(w_hbm.at[wid, pl.ds(t * K, K)], bufs[2], semi).wait()

    fire_idx(0, idxbufs[0])

    def zfill(i, carry):
        for g in range(H // 16):
            rows0[i, pl.ds(g * 16, 16)] = jnp.zeros((16,), jnp.float32)
        return carry
    lax.fori_loop(0, ZROWS, zfill, 0)
    for k in range(RPS // ZROWS):
        pltpu.sync_copy(rows0, accum.at[pl.ds(s * RPS + k * ZROWS, ZROWS)])
    plsc.subcore_barrier()

    for t in range(NB):
        srcb, dstb, wb = idxbufs[t % 2]
        drain_idx(t, idxbufs[t % 2])
        if t + 1 < NB:
            fire_idx(t + 1, idxbufs[(t + 1) % 2])

        for b in range(2):
            rows, sem = rowbufs[b]
            pltpu.async_copy(x_hbm.at[srcb.at[b]], rows, sem)

        def chunk_pair(jj, carry):
            for b in range(2):
                j = jj * 2 + b
                rows, sem = rowbufs[b]
                pltpu.make_async_copy(x_hbm.at[srcb.at[j]], rows, sem).wait()

                def scale(eg, inner):
                    ww = wb[j, pl.ds(eg * 16, 16)]
                    for i in range(16):
                        w = ww[i]
                        e = eg * 16 + i
                        for g in range(H // 16):
                            sl = pl.ds(g * 16, 16)
                            rows[e, sl] = rows[e, sl] * w
                    return inner
                lax.fori_loop(0, CHUNK // 16, scale, 0)

                pltpu.sync_copy(rows, accum.at[dstb.at[j]], add=True)

                @pl.when(j + 2 < K)
                def _():
                    pltpu.async_copy(x_hbm.at[srcb.at[j + 2]], rows, sem)
            return carry
        lax.fori_loop(0, K // 2, chunk_pair, 0)

    plsc.subcore_barrier()
    base = s * RPS
    pltpu.sync_copy(accum.at[pl.ds(base, RPS)], out_hbm.at[c, pl.ds(base, RPS)])


_spmm = pl.kernel(
    _spmm_body,
    out_type=jax.ShapeDtypeStruct((NC, NPAD, H), jnp.float32),
    mesh=plsc.VectorSubcoreMesh(core_axis_name="c", subcore_axis_name="s"),
    scratch_types=[
        pltpu.VMEM((K, CHUNK), jnp.int32),
        pltpu.VMEM((K, CHUNK), jnp.int32),
        pltpu.VMEM((K, CHUNK), jnp.int32),
        pltpu.VMEM((K, CHUNK), jnp.int32),
        pltpu.VMEM((K, CHUNK), jnp.float32),
        pltpu.VMEM((K, CHUNK), jnp.float32),
        pltpu.VMEM((CHUNK, H), jnp.float32),
        pltpu.VMEM((CHUNK, H), jnp.float32),
        pltpu.VMEM_SHARED((NPAD, H), jnp.float32),
        pltpu.SemaphoreType.DMA,
        pltpu.SemaphoreType.DMA,
        pltpu.SemaphoreType.DMA,
    ],
)

_BLK = 1024


def _add2_body(a_ref, b_ref, o_ref):
    o_ref[...] = a_ref[...] + b_ref[...]


def _combine(p):
    return pl.pallas_call(
        _add2_body,
        out_shape=jax.ShapeDtypeStruct((NPAD, H), jnp.float32),
        grid=(NPAD // _BLK,),
        in_specs=[pl.BlockSpec((_BLK, H), lambda i: (i, 0)),
                  pl.BlockSpec((_BLK, H), lambda i: (i, 0))],
        out_specs=pl.BlockSpec((_BLK, H), lambda i: (i, 0)),
    )(p[0], p[1])


def _mean_body(e_ref, x1_ref, x2_ref, pa_ref, pb_ref, o_ref):
    o_ref[...] = 0.25 * (e_ref[...] + x1_ref[...] + x2_ref[...]
                         + pa_ref[...] + pb_ref[...])


def _mean(ego, x1, x2, pa, pb):
    spec = pl.BlockSpec((_BLK, H), lambda i: (i, 0))
    return pl.pallas_call(
        _mean_body,
        out_shape=jax.ShapeDtypeStruct((NPAD, H), jnp.float32),
        grid=(NPAD // _BLK,),
        in_specs=[spec] * 5,
        out_specs=spec,
    )(ego, x1, x2, pa, pb)


def kernel(adj_indices, adj_values, user_emb, item_emb):
    dst = adj_indices[0].astype(jnp.int32)
    src = adj_indices[1].astype(jnp.int32)
    w = adj_values.astype(jnp.float32)
    pad = EP - E
    src3 = jnp.pad(src, (0, pad)).reshape(NW, CH, CHUNK)
    dst3 = jnp.pad(dst, (0, pad)).reshape(NW, CH, CHUNK)
    w3 = jnp.pad(w, (0, pad)).reshape(NW, CH, CHUNK)

    ego = jnp.pad(jnp.concatenate([user_emb, item_emb], axis=0),
                  ((0, NPAD - N), (0, 0)))
    p1 = _spmm(ego, src3, dst3, w3)
    x1 = _combine(p1)
    p2 = _spmm(x1, src3, dst3, w3)
    x2 = _combine(p2)
    p3 = _spmm(x2, src3, dst3, w3)
    final = _mean(ego, x1, x2, p3[0], p3[1])
    return final[:N_USERS], final[N_USERS:N]


# speedup vs baseline: 1.0518x; 1.0518x over previous
import jax
import jax.numpy as jnp
from jax import lax
from jax.experimental import pallas as pl
from jax.experimental.pallas import tpu as pltpu
from jax.experimental.pallas import tpu_sc as plsc

N_USERS = 5000
N_ITEMS = 5000
N = N_USERS + N_ITEMS
H = 128
E = 320000

NC = 2
NS = 16
NW = NC * NS
CHUNK = 64
CH = 160
EP = NW * CH * CHUNK
NPAD = 10240
ZROWS = 128
RPS = NPAD // NS

NBUF = 5
GD = 2
K = 8
NB = CH // K


def _spmm_body(x_hbm, src_hbm, dst_hbm, w_hbm, out_hbm,
               srcb, dstb, wb,
               r0, r1, r2, r3, r4, accum,
               g0, g1, g2, g3, g4,
               s0, s1, s2, s3, s4, semi):
    c = lax.axis_index("c")
    s = lax.axis_index("s")
    wid = s * NC + c

    rows = (r0, r1, r2, r3, r4)
    gsem = (g0, g1, g2, g3, g4)
    ssem = (s0, s1, s2, s3, s4)

    def fire_idx(blk):
        half = pl.ds(lax.bitwise_and(blk, 1) * K, K)
        sl = pl.ds(blk * K, K)
        pltpu.async_copy(src_hbm.at[wid, sl], srcb.at[half], semi)
        pltpu.async_copy(dst_hbm.at[wid, sl], dstb.at[half], semi)
        pltpu.async_copy(w_hbm.at[wid, sl], wb.at[half], semi)

    def drain_idx(blk):
        half = pl.ds(lax.bitwise_and(blk, 1) * K, K)
        sl = pl.ds(blk * K, K)
        pltpu.make_async_copy(src_hbm.at[wid, sl], srcb.at[half], semi).wait()
        pltpu.make_async_copy(dst_hbm.at[wid, sl], dstb.at[half], semi).wait()
        pltpu.make_async_copy(w_hbm.at[wid, sl], wb.at[half], semi).wait()

    fire_idx(0)

    def zfill(i, carry):
        for g in range(H // 16):
            r0[i, pl.ds(g * 16, 16)] = jnp.zeros((16,), jnp.float32)
        return carry
    lax.fori_loop(0, CHUNK, zfill, 0)
    for k in range(RPS // CHUNK):
        pltpu.sync_copy(r0, accum.at[pl.ds(s * RPS + k * CHUNK, CHUNK)])
    plsc.subcore_barrier()

    drain_idx(0)

    for j in range(GD):
        pltpu.async_copy(x_hbm.at[srcb.at[j]], rows[j], gsem[j])

    def block(jj, carry):
        for b in range(NBUF):
            j = jj * NBUF + b
            jm = lax.bitwise_and(j, 2 * K - 1)
            rb = rows[b]
            pltpu.make_async_copy(x_hbm.at[srcb.at[jm]], rb, gsem[b]).wait()

            def scale(eg, inner):
                ww = wb[jm, pl.ds(eg * 16, 16)]
                for i in range(16):
                    w = ww[i]
                    e = eg * 16 + i
                    for g in range(H // 16):
                        sl = pl.ds(g * 16, 16)
                        rb[e, sl] = rb[e, sl] * w
                return inner
            lax.fori_loop(0, CHUNK // 16, scale, 0)

            pltpu.async_copy(rb, accum.at[dstb.at[jm]], ssem[b], add=True)

            @pl.when(lax.bitwise_and(j, K - 1) == 0)
            def _():
                @pl.when(j + K < CH)
                def _():
                    fire_idx(lax.shift_right_logical(j, 3) + 1)

            @pl.when(lax.bitwise_and(j, K - 1) == K - GD)
            def _():
                @pl.when(j + GD < CH)
                def _():
                    drain_idx(lax.shift_right_logical(j, 3) + 1)

            bn = (b + GD) % NBUF

            @pl.when(j + GD < CH)
            def _():
                @pl.when(j >= NBUF - GD)
                def _():
                    jo = lax.bitwise_and(j - (NBUF - GD), 2 * K - 1)
                    pltpu.make_async_copy(
                        rows[bn], accum.at[dstb.at[jo]], ssem[bn]).wait()
                jn = lax.bitwise_and(j + GD, 2 * K - 1)
                pltpu.async_copy(x_hbm.at[srcb.at[jn]], rows[bn], gsem[bn])
        return carry
    lax.fori_loop(0, CH // NBUF, block, 0)

    for b in range(NBUF):
        jo = (CH - NBUF + b) % (2 * K)
        pltpu.make_async_copy(rows[b], accum.at[dstb.at[jo]], ssem[b]).wait()

    plsc.subcore_barrier()
    base = s * RPS
    pltpu.sync_copy(accum.at[pl.ds(base, RPS)], out_hbm.at[c, pl.ds(base, RPS)])


_spmm = pl.kernel(
    _spmm_body,
    out_type=jax.ShapeDtypeStruct((NC, NPAD, H), jnp.float32),
    mesh=plsc.VectorSubcoreMesh(core_axis_name="c", subcore_axis_name="s"),
    scratch_types=[
        pltpu.VMEM((2 * K, CHUNK), jnp.int32),
        pltpu.VMEM((2 * K, CHUNK), jnp.int32),
        pltpu.VMEM((2 * K, CHUNK), jnp.float32),
    ] + [pltpu.VMEM((CHUNK, H), jnp.float32)] * NBUF + [
        pltpu.VMEM_SHARED((NPAD, H), jnp.float32),
    ] + [pltpu.SemaphoreType.DMA] * (2 * NBUF + 1),
)

_BLK = 1024


def _add2_body(a_ref, b_ref, o_ref):
    o_ref[...] = a_ref[...] + b_ref[...]


def _combine(p):
    return pl.pallas_call(
        _add2_body,
        out_shape=jax.ShapeDtypeStruct((NPAD, H), jnp.float32),
        grid=(NPAD // _BLK,),
        in_specs=[pl.BlockSpec((_BLK, H), lambda i: (i, 0)),
                  pl.BlockSpec((_BLK, H), lambda i: (i, 0))],
        out_specs=pl.BlockSpec((_BLK, H), lambda i: (i, 0)),
    )(p[0], p[1])


def _mean_body(e_ref, x1_ref, x2_ref, pa_ref, pb_ref, o_ref):
    o_ref[...] = 0.25 * (e_ref[...] + x1_ref[...] + x2_ref[...]
                         + pa_ref[...] + pb_ref[...])


def _mean(ego, x1, x2, pa, pb):
    spec = pl.BlockSpec((_BLK, H), lambda i: (i, 0))
    return pl.pallas_call(
        _mean_body,
        out_shape=jax.ShapeDtypeStruct((NPAD, H), jnp.float32),
        grid=(NPAD // _BLK,),
        in_specs=[spec] * 5,
        out_specs=spec,
    )(ego, x1, x2, pa, pb)


def kernel(adj_indices, adj_values, user_emb, item_emb):
    dst = adj_indices[0].astype(jnp.int32)
    src = adj_indices[1].astype(jnp.int32)
    w = adj_values.astype(jnp.float32)
    pad = EP - E
    src3 = jnp.pad(src, (0, pad)).reshape(NW, CH, CHUNK)
    dst3 = jnp.pad(dst, (0, pad)).reshape(NW, CH, CHUNK)
    w3 = jnp.pad(w, (0, pad)).reshape(NW, CH, CHUNK)

    ego = jnp.pad(jnp.concatenate([user_emb, item_emb], axis=0),
                  ((0, NPAD - N), (0, 0)))
    p1 = _spmm(ego, src3, dst3, w3)
    x1 = _combine(p1)
    p2 = _spmm(x1, src3, dst3, w3)
    x2 = _combine(p2)
    p3 = _spmm(x2, src3, dst3, w3)
    final = _mean(ego, x1, x2, p3[0], p3[1])
    return final[:N_USERS], final[N_USERS:N]


# speedup vs baseline: 1.1406x; 1.0845x over previous
import jax
import jax.numpy as jnp
from jax import lax
from jax.experimental import pallas as pl
from jax.experimental.pallas import tpu as pltpu
from jax.experimental.pallas import tpu_sc as plsc

N_USERS = 5000
N_ITEMS = 5000
N = N_USERS + N_ITEMS
H = 128
E = 320000

NC = 2
NS = 16
NW = NC * NS
CHUNK = 64
CH0 = 240
CH1 = 80
CHT = CH0 + CH1
EP = NS * CHT * CHUNK
NPAD = 10240
ZROWS = 128
RPS = NPAD // NS

NBUF = 5
GD = 2
K = 8


def _spmm_body(x_hbm, src_hbm, dst_hbm, w_hbm, out_hbm,
               srcb, dstb, wb,
               r0, r1, r2, r3, r4, accum,
               g0, g1, g2, g3, g4,
               s0, s1, s2, s3, s4, semi):
    c = lax.axis_index("c")
    s = lax.axis_index("s")
    ch = lax.select(c == 0, jnp.int32(CH0), jnp.int32(CH1))
    nblk = lax.select(c == 0, jnp.int32(CH0 // NBUF), jnp.int32(CH1 // NBUF))

    rows = (r0, r1, r2, r3, r4)
    gsem = (g0, g1, g2, g3, g4)
    ssem = (s0, s1, s2, s3, s4)

    def fire_idx(blk):
        half = pl.ds(lax.bitwise_and(blk, 1) * K, K)
        sl = pl.ds(blk * K, K)
        pltpu.async_copy(src_hbm.at[c, s, sl], srcb.at[half], semi)
        pltpu.async_copy(dst_hbm.at[c, s, sl], dstb.at[half], semi)
        pltpu.async_copy(w_hbm.at[c, s, sl], wb.at[half], semi)

    def drain_idx(blk):
        half = pl.ds(lax.bitwise_and(blk, 1) * K, K)
        sl = pl.ds(blk * K, K)
        pltpu.make_async_copy(src_hbm.at[c, s, sl], srcb.at[half], semi).wait()
        pltpu.make_async_copy(dst_hbm.at[c, s, sl], dstb.at[half], semi).wait()
        pltpu.make_async_copy(w_hbm.at[c, s, sl], wb.at[half], semi).wait()

    fire_idx(0)

    def zfill(i, carry):
        for g in range(H // 16):
            r0[i, pl.ds(g * 16, 16)] = jnp.zeros((16,), jnp.float32)
        return carry
    lax.fori_loop(0, CHUNK, zfill, 0)
    for k in range(RPS // CHUNK):
        pltpu.sync_copy(r0, accum.at[pl.ds(s * RPS + k * CHUNK, CHUNK)])
    plsc.subcore_barrier()

    drain_idx(0)

    for j in range(GD):
        pltpu.async_copy(x_hbm.at[srcb.at[j]], rows[j], gsem[j])

    def block(jj, carry):
        for b in range(NBUF):
            j = jj * NBUF + b
            jm = lax.bitwise_and(j, 2 * K - 1)
            rb = rows[b]
            pltpu.make_async_copy(x_hbm.at[srcb.at[jm]], rb, gsem[b]).wait()

            def scale(eg, inner):
                ww = wb[jm, pl.ds(eg * 16, 16)]
                for i in range(16):
                    w = ww[i]
                    e = eg * 16 + i
                    for g in range(H // 16):
                        sl = pl.ds(g * 16, 16)
                        rb[e, sl] = rb[e, sl] * w
                return inner
            lax.fori_loop(0, CHUNK // 16, scale, 0)

            pltpu.async_copy(rb, accum.at[dstb.at[jm]], ssem[b], add=True)

            @pl.when(lax.bitwise_and(j, K - 1) == 0)
            def _():
                @pl.when(j + K < ch)
                def _():
                    fire_idx(lax.shift_right_logical(j, 3) + 1)

            @pl.when(lax.bitwise_and(j, K - 1) == K - GD)
            def _():
                @pl.when(j + GD < ch)
                def _():
                    drain_idx(lax.shift_right_logical(j, 3) + 1)

            bn = (b + GD) % NBUF

            @pl.when(j + GD < ch)
            def _():
                @pl.when(j >= NBUF - GD)
                def _():
                    jo = lax.bitwise_and(j - (NBUF - GD), 2 * K - 1)
                    pltpu.make_async_copy(
                        rows[bn], accum.at[dstb.at[jo]], ssem[bn]).wait()
                jn = lax.bitwise_and(j + GD, 2 * K - 1)
                pltpu.async_copy(x_hbm.at[srcb.at[jn]], rows[bn], gsem[bn])
        return carry
    lax.fori_loop(0, nblk, block, 0)

    for b in range(NBUF):
        jo = lax.bitwise_and(ch - NBUF + b, 2 * K - 1)
        pltpu.make_async_copy(rows[b], accum.at[dstb.at[jo]], ssem[b]).wait()

    plsc.subcore_barrier()
    base = s * RPS
    pltpu.sync_copy(accum.at[pl.ds(base, RPS)], out_hbm.at[c, pl.ds(base, RPS)])


_spmm = pl.kernel(
    _spmm_body,
    out_type=jax.ShapeDtypeStruct((NC, NPAD, H), jnp.float32),
    mesh=plsc.VectorSubcoreMesh(core_axis_name="c", subcore_axis_name="s"),
    scratch_types=[
        pltpu.VMEM((2 * K, CHUNK), jnp.int32),
        pltpu.VMEM((2 * K, CHUNK), jnp.int32),
        pltpu.VMEM((2 * K, CHUNK), jnp.float32),
    ] + [pltpu.VMEM((CHUNK, H), jnp.float32)] * NBUF + [
        pltpu.VMEM_SHARED((NPAD, H), jnp.float32),
    ] + [pltpu.SemaphoreType.DMA] * (2 * NBUF + 1),
)

_BLK = 1024


def _add2_body(a_ref, b_ref, o_ref):
    o_ref[...] = a_ref[...] + b_ref[...]


def _combine(p):
    return pl.pallas_call(
        _add2_body,
        out_shape=jax.ShapeDtypeStruct((NPAD, H), jnp.float32),
        grid=(NPAD // _BLK,),
        in_specs=[pl.BlockSpec((_BLK, H), lambda i: (i, 0)),
                  pl.BlockSpec((_BLK, H), lambda i: (i, 0))],
        out_specs=pl.BlockSpec((_BLK, H), lambda i: (i, 0)),
    )(p[0], p[1])


def _mean_body(e_ref, x1_ref, x2_ref, pa_ref, pb_ref, o_ref):
    o_ref[...] = 0.25 * (e_ref[...] + x1_ref[...] + x2_ref[...]
                         + pa_ref[...] + pb_ref[...])


def _mean(ego, x1, x2, pa, pb):
    spec = pl.BlockSpec((_BLK, H), lambda i: (i, 0))
    return pl.pallas_call(
        _mean_body,
        out_shape=jax.ShapeDtypeStruct((NPAD, H), jnp.float32),
        grid=(NPAD // _BLK,),
        in_specs=[spec] * 5,
        out_specs=spec,
    )(ego, x1, x2, pa, pb)


def _pack(a):
    a = jnp.pad(a, (0, EP - E))
    e0 = NS * CH0 * CHUNK
    a0 = a[:e0].reshape(NS, CH0, CHUNK)
    a1 = a[e0:].reshape(NS, CH1, CHUNK)
    a1 = jnp.pad(a1, ((0, 0), (0, CH0 - CH1), (0, 0)))
    return jnp.stack([a0, a1])


def kernel(adj_indices, adj_values, user_emb, item_emb):
    dst = adj_indices[0].astype(jnp.int32)
    src = adj_indices[1].astype(jnp.int32)
    w = adj_values.astype(jnp.float32)
    src3 = _pack(src)
    dst3 = _pack(dst)
    w3 = _pack(w)

    ego = jnp.pad(jnp.concatenate([user_emb, item_emb], axis=0),
                  ((0, NPAD - N), (0, 0)))
    p1 = _spmm(ego, src3, dst3, w3)
    x1 = _combine(p1)
    p2 = _spmm(x1, src3, dst3, w3)
    x2 = _combine(p2)
    p3 = _spmm(x2, src3, dst3, w3)
    final = _mean(ego, x1, x2, p3[0], p3[1])
    return final[:N_USERS], final[N_USERS:N]
